# Initial kernel scaffold; baseline (speedup 1.0000x reference)
import jax
import jax.numpy as jnp
from jax.experimental import pallas as pl


def kernel(edge_index, edge_type, emb, bases1, comp1, root1, bias1, bases2, comp2, root2, bias2):
    raise NotImplementedError("write your pallas kernel here")


# speedup vs baseline: 1.9631x; 1.9631x over previous
import functools

import jax
import jax.numpy as jnp
from jax import lax
from jax.experimental import pallas as pl
from jax.experimental.pallas import tpu as pltpu
from jax.experimental.pallas import tpu_sc as plsc

N = 10000
R = 32
D = 128
NB = 30
E = 320000

NC = 2
NS = 16
NW = NC * NS
K = 80

_ROWS_T = 1000
_CNT_T = (R * N) // NS
_EC = E // NS
_EA = E // NW
_ZC = 2000


def _w_body(comp_ref, bases_ref, out_ref):
    out_ref[...] = jnp.dot(comp_ref[...], bases_ref[...],
                           preferred_element_type=jnp.float32)


def _w_matmul(comp, bases_flat):
    return pl.pallas_call(
        _w_body,
        out_shape=jax.ShapeDtypeStruct((R, D * D), jnp.float32),
    )(comp, bases_flat)


def _y_body(x_ref, w_ref, out_ref):
    out_ref[...] = jnp.dot(x_ref[...], w_ref[...],
                           preferred_element_type=jnp.float32)


def _y_matmul(x, w2):
    return pl.pallas_call(
        _y_body,
        grid=(R,),
        in_specs=[
            pl.BlockSpec((N, D), lambda r: (0, 0)),
            pl.BlockSpec((D, D), lambda r: (r, 0)),
        ],
        out_specs=pl.BlockSpec((N, D), lambda r: (r, 0)),
        out_shape=jax.ShapeDtypeStruct((R * N, D), jnp.float32),
    )(x, w2)


_BN = 2000


def _final_body(p0_ref, p1_ref, x_ref, root_ref, bias_ref, out_ref):
    acc = p0_ref[...] + p1_ref[...]
    acc = acc + jnp.dot(x_ref[...], root_ref[...],
                        preferred_element_type=jnp.float32)
    out_ref[...] = jnp.maximum(acc + bias_ref[...], 0.0)


def _final(parts, x, root, bias2d):
    nb = N // _BN
    return pl.pallas_call(
        _final_body,
        grid=(nb,),
        in_specs=[
            pl.BlockSpec((_BN, D), lambda i: (i, 0)),
            pl.BlockSpec((_BN, D), lambda i, _nb=nb: (i + _nb, 0)),
            pl.BlockSpec((_BN, D), lambda i: (i, 0)),
            pl.BlockSpec((D, D), lambda i: (0, 0)),
            pl.BlockSpec((1, D), lambda i: (0, 0)),
        ],
        out_specs=pl.BlockSpec((_BN, D), lambda i: (i, 0)),
        out_shape=jax.ShapeDtypeStruct((N, D), jnp.float32),
    )(parts, parts, x, root, bias2d)


def _sc_body(seg_hbm, gsrc_hbm, dst_hbm, y_hbm, zf_hbm, out_hbm,
             counts_sh, acc_sh,
             seg_v, gidx_v, didx_v, cnt_v, inv_v, ones_v, zcnt_v, rows_v, sem):
    c = lax.axis_index("c")
    s = lax.axis_index("s")

    @pl.when(s < N // _ROWS_T)
    def _():
        pltpu.sync_copy(zf_hbm.at[pl.ds(s * _ROWS_T, _ROWS_T)],
                        acc_sh.at[pl.ds(s * _ROWS_T, _ROWS_T)])

    def zero16(i, carry):
        zcnt_v[pl.ds(i * 16, 16)] = jnp.zeros((16,), jnp.int32)
        return carry

    lax.fori_loop(0, _ZC // 16, zero16, 0)

    def zcopy(i, carry):
        pltpu.sync_copy(zcnt_v, counts_sh.at[pl.ds(s * _CNT_T + i * _ZC, _ZC)])
        return carry

    lax.fori_loop(0, _CNT_T // _ZC, zcopy, 0)
    for u in range(K // 16):
        ones_v[pl.ds(u * 16, 16)] = jnp.ones((16,), jnp.int32)
    plsc.subcore_barrier()

    def count_chunk(i, carry):
        off = s * _EC + i * K
        pltpu.sync_copy(seg_hbm.at[pl.ds(off, K)], seg_v)
        pltpu.sync_copy(ones_v, counts_sh.at[seg_v], add=True)
        return carry

    lax.fori_loop(0, _EC // K, count_chunk, 0)
    plsc.subcore_barrier()

    def agg_chunk(i, carry):
        off = (s * NC + c) * _EA + i * K
        pltpu.sync_copy(gsrc_hbm.at[pl.ds(off, K)], gidx_v)
        cp = pltpu.async_copy(y_hbm.at[gidx_v], rows_v, sem)
        pltpu.sync_copy(seg_hbm.at[pl.ds(off, K)], seg_v)
        pltpu.sync_copy(dst_hbm.at[pl.ds(off, K)], didx_v)
        pltpu.sync_copy(counts_sh.at[seg_v], cnt_v)
        for u in range(K // 16):
            cc = cnt_v[pl.ds(u * 16, 16)]
            inv_v[pl.ds(u * 16, 16)] = 1.0 / cc.astype(jnp.float32)
        cp.wait()

        def scale_row(j, carry2):
            splat = jnp.zeros((16,), jnp.int32) + j
            sv = plsc.load_gather(inv_v, [splat])
            for u in range(D // 16):
                rows_v[j, pl.ds(u * 16, 16)] = rows_v[j, pl.ds(u * 16, 16)] * sv
            return carry2

        lax.fori_loop(0, K, scale_row, 0)
        pltpu.sync_copy(rows_v, acc_sh.at[didx_v], add=True)
        return carry

    lax.fori_loop(0, _EA // K, agg_chunk, 0)
    plsc.subcore_barrier()

    @pl.when(s < N // _ROWS_T)
    def _():
        pltpu.sync_copy(acc_sh.at[pl.ds(s * _ROWS_T, _ROWS_T)],
                        out_hbm.at[pl.ds(c * N + s * _ROWS_T, _ROWS_T)])


@functools.partial(jax.jit, static_argnums=())
def _sc_agg(seg, gsrc, dst, y, zf):
    mesh = plsc.VectorSubcoreMesh(core_axis_name="c", subcore_axis_name="s")
    f = pl.kernel(
        _sc_body,
        out_type=jax.ShapeDtypeStruct((2 * N, D), jnp.float32),
        mesh=mesh,
        compiler_params=pltpu.CompilerParams(needs_layout_passes=False),
        scratch_types=[
            pltpu.VMEM_SHARED((R * N,), jnp.int32),
            pltpu.VMEM_SHARED((N, D), jnp.float32),
            pltpu.VMEM((K,), jnp.int32),
            pltpu.VMEM((K,), jnp.int32),
            pltpu.VMEM((K,), jnp.int32),
            pltpu.VMEM((K,), jnp.int32),
            pltpu.VMEM((K,), jnp.float32),
            pltpu.VMEM((K,), jnp.int32),
            pltpu.VMEM((_ZC,), jnp.int32),
            pltpu.VMEM((K, D), jnp.float32),
            pltpu.SemaphoreType.DMA,
        ],
    )
    return f(seg, gsrc, dst, y, zf)


def _layer(x, seg, gsrc, dst, zf, bases, comp, root, bias):
    wall = _w_matmul(comp, bases.reshape(NB, D * D))
    y = _y_matmul(x, wall.reshape(R * D, D))
    parts = _sc_agg(seg, gsrc, dst, y, zf)
    return _final(parts, x, root, bias.reshape(1, D))


def kernel(edge_index, edge_type, emb, bases1, comp1, root1, bias1,
           bases2, comp2, root2, bias2):
    src = edge_index[0]
    dst = edge_index[1]
    gsrc = edge_type * N + src
    seg = edge_type * N + dst
    zf = jnp.zeros((N, D), jnp.float32)
    x1 = _layer(emb, seg, gsrc, dst, zf, bases1, comp1, root1, bias1)
    x2 = _layer(x1, seg, gsrc, dst, zf, bases2, comp2, root2, bias2)
    return x2


# speedup vs baseline: 4.2750x; 2.1777x over previous
import functools

import jax
import jax.numpy as jnp
from jax import lax
from jax.experimental import pallas as pl
from jax.experimental.pallas import tpu as pltpu
from jax.experimental.pallas import tpu_sc as plsc

N = 10000
R = 32
D = 128
NB = 30
E = 320000

NC = 2
NS = 16
NW = NC * NS
K = 80

_ROWS_T = 1000
_CNT_T = (R * N) // NS
_EC = E // NS
_EA = E // NW
_ZC = 2000
_NCH = _EA // K


def _w_body(comp_ref, bases_ref, out_ref):
    out_ref[...] = jnp.dot(comp_ref[...], bases_ref[...],
                           preferred_element_type=jnp.float32)


def _w_matmul(comp, bases_flat):
    return pl.pallas_call(
        _w_body,
        out_shape=jax.ShapeDtypeStruct((R, D * D), jnp.float32),
    )(comp, bases_flat)


def _y_body(x_ref, w_ref, out_ref):
    out_ref[...] = jnp.dot(x_ref[...], w_ref[...],
                           preferred_element_type=jnp.float32)


def _y_matmul(x, w2):
    return pl.pallas_call(
        _y_body,
        grid=(R,),
        in_specs=[
            pl.BlockSpec((N, D), lambda r: (0, 0)),
            pl.BlockSpec((D, D), lambda r: (r, 0)),
        ],
        out_specs=pl.BlockSpec((N, D), lambda r: (r, 0)),
        out_shape=jax.ShapeDtypeStruct((R * N, D), jnp.float32),
    )(x, w2)


_BN = 2000


def _final_body(p0_ref, p1_ref, x_ref, root_ref, bias_ref, out_ref):
    acc = p0_ref[...] + p1_ref[...]
    acc = acc + jnp.dot(x_ref[...], root_ref[...],
                        preferred_element_type=jnp.float32)
    out_ref[...] = jnp.maximum(acc + bias_ref[...], 0.0)


def _final(parts, x, root, bias2d):
    nb = N // _BN
    return pl.pallas_call(
        _final_body,
        grid=(nb,),
        in_specs=[
            pl.BlockSpec((_BN, D), lambda i: (i, 0)),
            pl.BlockSpec((_BN, D), lambda i, _nb=nb: (i + _nb, 0)),
            pl.BlockSpec((_BN, D), lambda i: (i, 0)),
            pl.BlockSpec((D, D), lambda i: (0, 0)),
            pl.BlockSpec((1, D), lambda i: (0, 0)),
        ],
        out_specs=pl.BlockSpec((_BN, D), lambda i: (i, 0)),
        out_shape=jax.ShapeDtypeStruct((N, D), jnp.float32),
    )(parts, parts, x, root, bias2d)


def _vcopy(dst_ref, src_ref, src_off, n):
    for u in range(n // 16):
        dst_ref[pl.ds(u * 16, 16)] = src_ref[pl.ds(src_off + u * 16, 16)]


def _deg_body(segc_hbm, segb_hbm, inv_hbm,
              counts_sh,
              segc_v, segb_v, seg_v, ones_v, cnt_v, invf_v, zcnt_v):
    c = lax.axis_index("c")
    s = lax.axis_index("s")
    wid = s * NC + c

    def zero16(i, carry):
        zcnt_v[pl.ds(i * 16, 16)] = jnp.zeros((16,), jnp.int32)
        return carry

    lax.fori_loop(0, _ZC // 16, zero16, 0)

    def zcopy(i, carry):
        pltpu.sync_copy(zcnt_v, counts_sh.at[pl.ds(s * _CNT_T + i * _ZC, _ZC)])
        return carry

    lax.fori_loop(0, _CNT_T // _ZC, zcopy, 0)
    for u in range(K // 16):
        ones_v[pl.ds(u * 16, 16)] = jnp.ones((16,), jnp.int32)
    plsc.subcore_barrier()

    pltpu.sync_copy(segc_hbm.at[pl.ds(s * _EC, _EC)], segc_v)

    def count_chunk(i, carry):
        _vcopy(seg_v, segc_v, i * K, K)
        pltpu.sync_copy(ones_v, counts_sh.at[seg_v], add=True)
        return carry

    lax.fori_loop(0, _EC // K, count_chunk, 0)
    plsc.subcore_barrier()

    pltpu.sync_copy(segb_hbm.at[pl.ds(wid * _EA, _EA)], segb_v)

    def inv_chunk(i, carry):
        _vcopy(seg_v, segb_v, i * K, K)
        pltpu.sync_copy(counts_sh.at[seg_v], cnt_v)
        for u in range(K // 16):
            cc = cnt_v[pl.ds(u * 16, 16)]
            invf_v[pl.ds(i * K + u * 16, 16)] = 1.0 / cc.astype(jnp.float32)
        return carry

    lax.fori_loop(0, _NCH, inv_chunk, 0)
    pltpu.sync_copy(invf_v, inv_hbm.at[pl.ds(wid * _EA, _EA)])


def _deg_kernel(seg):
    mesh = plsc.VectorSubcoreMesh(core_axis_name="c", subcore_axis_name="s")
    f = pl.kernel(
        _deg_body,
        out_type=jax.ShapeDtypeStruct((E,), jnp.float32),
        mesh=mesh,
        compiler_params=pltpu.CompilerParams(needs_layout_passes=False),
        scratch_types=[
            pltpu.VMEM_SHARED((R * N,), jnp.int32),
            pltpu.VMEM((_EC,), jnp.int32),
            pltpu.VMEM((_EA,), jnp.int32),
            pltpu.VMEM((K,), jnp.int32),
            pltpu.VMEM((K,), jnp.int32),
            pltpu.VMEM((K,), jnp.int32),
            pltpu.VMEM((_EA,), jnp.float32),
            pltpu.VMEM((_ZC,), jnp.int32),
        ],
    )
    return f(seg, seg)


def _agg_body(gsrc_hbm, dst_hbm, inv_hbm, y_hbm, zf_hbm, out_hbm,
              acc_sh,
              gidxf_v, didxf_v, invf_v, didxA, didxB,
              rowsA, rowsB, semgA, semgB, semsA, semsB):
    c = lax.axis_index("c")
    s = lax.axis_index("s")
    wid = s * NC + c

    @pl.when(s < N // _ROWS_T)
    def _():
        pltpu.sync_copy(zf_hbm.at[pl.ds(s * _ROWS_T, _ROWS_T)],
                        acc_sh.at[pl.ds(s * _ROWS_T, _ROWS_T)])

    pltpu.sync_copy(gsrc_hbm.at[pl.ds(wid * _EA, _EA)], gidxf_v)
    pltpu.sync_copy(dst_hbm.at[pl.ds(wid * _EA, _EA)], didxf_v)
    pltpu.sync_copy(inv_hbm.at[pl.ds(wid * _EA, _EA)], invf_v)
    plsc.subcore_barrier()

    def fire_gather(i, rows_v, semg):
        pltpu.async_copy(y_hbm.at[gidxf_v.at[pl.ds(i * K, K)]], rows_v, semg)

    def wait_gather(rows_v, semg):
        pltpu.make_async_copy(y_hbm.at[gidxf_v.at[pl.ds(0, K)]], rows_v,
                              semg).wait()

    def wait_scatter(rows_v, didx_v, sems):
        pltpu.make_async_copy(rows_v, acc_sh.at[didx_v], sems).wait()

    def handle(i, rows_v, didx_v, semg, sems):
        wait_gather(rows_v, semg)

        def scale_row(j, carry):
            splat = jnp.zeros((16,), jnp.int32) + (i * K + j)
            sv = plsc.load_gather(invf_v, [splat])
            for u in range(D // 16):
                rows_v[j, pl.ds(u * 16, 16)] = (
                    rows_v[j, pl.ds(u * 16, 16)] * sv)
            return carry

        lax.fori_loop(0, K, scale_row, 0)
        _vcopy(didx_v, didxf_v, i * K, K)
        pltpu.async_copy(rows_v, acc_sh.at[didx_v], sems, add=True)

    fire_gather(0, rowsA, semgA)
    fire_gather(1, rowsB, semgB)

    def pair(i2, carry):
        handle(2 * i2, rowsA, didxA, semgA, semsA)

        @pl.when(i2 < (_NCH - 1) // 2)
        def _():
            wait_scatter(rowsA, didxA, semsA)
            fire_gather(2 * i2 + 2, rowsA, semgA)

        handle(2 * i2 + 1, rowsB, didxB, semgB, semsB)

        @pl.when(i2 < (_NCH - 3) // 2)
        def _():
            wait_scatter(rowsB, didxB, semsB)
            fire_gather(2 * i2 + 3, rowsB, semgB)

        return carry

    lax.fori_loop(0, (_NCH - 1) // 2, pair, 0)
    wait_scatter(rowsB, didxB, semsB)
    handle(_NCH - 1, rowsA, didxA, semgA, semsA)
    wait_scatter(rowsA, didxA, semsA)
    plsc.subcore_barrier()

    @pl.when(s < N // _ROWS_T)
    def _():
        pltpu.sync_copy(acc_sh.at[pl.ds(s * _ROWS_T, _ROWS_T)],
                        out_hbm.at[pl.ds(c * N + s * _ROWS_T, _ROWS_T)])


def _agg_kernel(gsrc, dst, inv, y, zf):
    mesh = plsc.VectorSubcoreMesh(core_axis_name="c", subcore_axis_name="s")
    f = pl.kernel(
        _agg_body,
        out_type=jax.ShapeDtypeStruct((2 * N, D), jnp.float32),
        mesh=mesh,
        compiler_params=pltpu.CompilerParams(needs_layout_passes=False),
        scratch_types=[
            pltpu.VMEM_SHARED((N, D), jnp.float32),
            pltpu.VMEM((_EA,), jnp.int32),
            pltpu.VMEM((_EA,), jnp.int32),
            pltpu.VMEM((_EA,), jnp.float32),
            pltpu.VMEM((K,), jnp.int32),
            pltpu.VMEM((K,), jnp.int32),
            pltpu.VMEM((K, D), jnp.float32),
            pltpu.VMEM((K, D), jnp.float32),
            pltpu.SemaphoreType.DMA,
            pltpu.SemaphoreType.DMA,
            pltpu.SemaphoreType.DMA,
            pltpu.SemaphoreType.DMA,
        ],
    )
    return f(gsrc, dst, inv, y, zf)


def _layer(x, gsrc, dst, inv, zf, bases, comp, root, bias):
    wall = _w_matmul(comp, bases.reshape(NB, D * D))
    y = _y_matmul(x, wall.reshape(R * D, D))
    parts = _agg_kernel(gsrc, dst, inv, y, zf)
    return _final(parts, x, root, bias.reshape(1, D))


def kernel(edge_index, edge_type, emb, bases1, comp1, root1, bias1,
           bases2, comp2, root2, bias2):
    src = edge_index[0]
    dst = edge_index[1]
    gsrc = edge_type * N + src
    seg = edge_type * N + dst
    zf = jnp.zeros((N, D), jnp.float32)
    inv = _deg_kernel(seg)
    x1 = _layer(emb, gsrc, dst, inv, zf, bases1, comp1, root1, bias1)
    x2 = _layer(x1, gsrc, dst, inv, zf, bases2, comp2, root2, bias2)
    return x2


# speedup vs baseline: 4.9430x; 1.1563x over previous
import functools

import jax
import jax.numpy as jnp
from jax import lax
from jax.experimental import pallas as pl
from jax.experimental.pallas import tpu as pltpu
from jax.experimental.pallas import tpu_sc as plsc

N = 10000
R = 32
D = 128
NB = 30
E = 320000

NC = 2
NS = 16
NW = NC * NS
K = 80

_ROWS_T = 1000
_CNT_T = (R * N) // NS
_EC = E // NS
_EA = E // NW
_ZC = 2000
_NCH = _EA // K


def _w_body(comp_ref, bases_ref, out_ref):
    out_ref[...] = jnp.dot(comp_ref[...], bases_ref[...],
                           preferred_element_type=jnp.float32)


def _w_matmul(comp, bases_flat):
    return pl.pallas_call(
        _w_body,
        out_shape=jax.ShapeDtypeStruct((R, D * D), jnp.float32),
    )(comp, bases_flat)


def _y_body(x_ref, w_ref, out_ref):
    out_ref[...] = jnp.dot(x_ref[...], w_ref[...],
                           preferred_element_type=jnp.float32)


def _y_matmul(x, w2):
    return pl.pallas_call(
        _y_body,
        grid=(R,),
        in_specs=[
            pl.BlockSpec((N, D), lambda r: (0, 0)),
            pl.BlockSpec((D, D), lambda r: (r, 0)),
        ],
        out_specs=pl.BlockSpec((N, D), lambda r: (r, 0)),
        out_shape=jax.ShapeDtypeStruct((R * N, D), jnp.float32),
    )(x, w2)


_BN = 2000


def _final_body(p0_ref, p1_ref, x_ref, root_ref, bias_ref, out_ref):
    acc = p0_ref[...] + p1_ref[...]
    acc = acc + jnp.dot(x_ref[...], root_ref[...],
                        preferred_element_type=jnp.float32)
    out_ref[...] = jnp.maximum(acc + bias_ref[...], 0.0)


def _final(parts, x, root, bias2d):
    nb = N // _BN
    return pl.pallas_call(
        _final_body,
        grid=(nb,),
        in_specs=[
            pl.BlockSpec((_BN, D), lambda i: (i, 0)),
            pl.BlockSpec((_BN, D), lambda i, _nb=nb: (i + _nb, 0)),
            pl.BlockSpec((_BN, D), lambda i: (i, 0)),
            pl.BlockSpec((D, D), lambda i: (0, 0)),
            pl.BlockSpec((1, D), lambda i: (0, 0)),
        ],
        out_specs=pl.BlockSpec((_BN, D), lambda i: (i, 0)),
        out_shape=jax.ShapeDtypeStruct((N, D), jnp.float32),
    )(parts, parts, x, root, bias2d)


def _vcopy(dst_ref, src_ref, src_off, n):
    for u in range(n // 16):
        dst_ref[pl.ds(u * 16, 16)] = src_ref[pl.ds(src_off + u * 16, 16)]


def _deg_body(segc_hbm, segb_hbm, inv_hbm,
              counts_sh,
              segc_v, segb_v, seg_v, ones_v, cnt_v, invf_v, zcnt_v):
    c = lax.axis_index("c")
    s = lax.axis_index("s")
    wid = s * NC + c

    def zero16(i, carry):
        zcnt_v[pl.ds(i * 16, 16)] = jnp.zeros((16,), jnp.int32)
        return carry

    lax.fori_loop(0, _ZC // 16, zero16, 0)

    def zcopy(i, carry):
        pltpu.sync_copy(zcnt_v, counts_sh.at[pl.ds(s * _CNT_T + i * _ZC, _ZC)])
        return carry

    lax.fori_loop(0, _CNT_T // _ZC, zcopy, 0)
    for u in range(K // 16):
        ones_v[pl.ds(u * 16, 16)] = jnp.ones((16,), jnp.int32)
    plsc.subcore_barrier()

    pltpu.sync_copy(segc_hbm.at[pl.ds(s * _EC, _EC)], segc_v)

    def count_chunk(i, carry):
        _vcopy(seg_v, segc_v, i * K, K)
        pltpu.sync_copy(ones_v, counts_sh.at[seg_v], add=True)
        return carry

    lax.fori_loop(0, _EC // K, count_chunk, 0)
    plsc.subcore_barrier()

    pltpu.sync_copy(segb_hbm.at[pl.ds(wid * _EA, _EA)], segb_v)

    def inv_chunk(i, carry):
        _vcopy(seg_v, segb_v, i * K, K)
        pltpu.sync_copy(counts_sh.at[seg_v], cnt_v)
        for u in range(K // 16):
            cc = cnt_v[pl.ds(u * 16, 16)]
            invf_v[pl.ds(i * K + u * 16, 16)] = 1.0 / cc.astype(jnp.float32)
        return carry

    lax.fori_loop(0, _NCH, inv_chunk, 0)
    pltpu.sync_copy(invf_v, inv_hbm.at[pl.ds(wid * _EA, _EA)])


def _deg_kernel(seg):
    mesh = plsc.VectorSubcoreMesh(core_axis_name="c", subcore_axis_name="s")
    f = pl.kernel(
        _deg_body,
        out_type=jax.ShapeDtypeStruct((E,), jnp.float32),
        mesh=mesh,
        compiler_params=pltpu.CompilerParams(needs_layout_passes=False),
        scratch_types=[
            pltpu.VMEM_SHARED((R * N,), jnp.int32),
            pltpu.VMEM((_EC,), jnp.int32),
            pltpu.VMEM((_EA,), jnp.int32),
            pltpu.VMEM((K,), jnp.int32),
            pltpu.VMEM((K,), jnp.int32),
            pltpu.VMEM((K,), jnp.int32),
            pltpu.VMEM((_EA,), jnp.float32),
            pltpu.VMEM((_ZC,), jnp.int32),
        ],
    )
    return f(seg, seg)


def _agg_body(gsrc_hbm, dst_hbm, inv_hbm, y_hbm, zf_hbm, out_hbm,
              acc_sh,
              gidxf_v, didxf_v, invf_v,
              rowsA, rowsB, semgA, semgB, semsA, semsB):
    c = lax.axis_index("c")
    s = lax.axis_index("s")
    wid = s * NC + c

    @pl.when(s < N // _ROWS_T)
    def _():
        pltpu.sync_copy(zf_hbm.at[pl.ds(s * _ROWS_T, _ROWS_T)],
                        acc_sh.at[pl.ds(s * _ROWS_T, _ROWS_T)])

    pltpu.sync_copy(gsrc_hbm.at[pl.ds(wid * _EA, _EA)], gidxf_v)
    pltpu.sync_copy(dst_hbm.at[pl.ds(wid * _EA, _EA)], didxf_v)
    pltpu.sync_copy(inv_hbm.at[pl.ds(wid * _EA, _EA)], invf_v)
    plsc.subcore_barrier()

    def fire_gather(i, rows_v, semg):
        pltpu.async_copy(y_hbm.at[gidxf_v.at[pl.ds(i * K, K)]], rows_v, semg)

    def wait_gather(rows_v, semg):
        pltpu.make_async_copy(y_hbm.at[gidxf_v.at[pl.ds(0, K)]], rows_v,
                              semg).wait()

    def wait_scatter(rows_v, sems):
        pltpu.make_async_copy(rows_v, acc_sh.at[didxf_v.at[pl.ds(0, K)]],
                              sems).wait()

    def handle(i, rows_v, semg, sems):
        wait_gather(rows_v, semg)

        def scale16(t, carry):
            iv = invf_v[pl.ds(i * K + t * 16, 16)]
            for j in range(16):
                sv = lax.gather(
                    iv, jnp.full((16, 1), j, jnp.int32),
                    lax.GatherDimensionNumbers(
                        offset_dims=(), collapsed_slice_dims=(0,),
                        start_index_map=(0,)),
                    slice_sizes=(1,),
                    mode=lax.GatherScatterMode.PROMISE_IN_BOUNDS)
                r = t * 16 + j
                for u in range(D // 16):
                    rows_v[r, pl.ds(u * 16, 16)] = (
                        rows_v[r, pl.ds(u * 16, 16)] * sv)
            return carry

        lax.fori_loop(0, K // 16, scale16, 0)
        pltpu.async_copy(rows_v, acc_sh.at[didxf_v.at[pl.ds(i * K, K)]],
                         sems, add=True)

    fire_gather(0, rowsA, semgA)
    fire_gather(1, rowsB, semgB)

    def pair(i2, carry):
        handle(2 * i2, rowsA, semgA, semsA)

        @pl.when(i2 < (_NCH - 1) // 2)
        def _():
            wait_scatter(rowsA, semsA)
            fire_gather(2 * i2 + 2, rowsA, semgA)

        handle(2 * i2 + 1, rowsB, semgB, semsB)

        @pl.when(i2 < (_NCH - 3) // 2)
        def _():
            wait_scatter(rowsB, semsB)
            fire_gather(2 * i2 + 3, rowsB, semgB)

        return carry

    lax.fori_loop(0, (_NCH - 1) // 2, pair, 0)
    wait_scatter(rowsB, semsB)
    handle(_NCH - 1, rowsA, semgA, semsA)
    wait_scatter(rowsA, semsA)
    plsc.subcore_barrier()

    @pl.when(s < N // _ROWS_T)
    def _():
        pltpu.sync_copy(acc_sh.at[pl.ds(s * _ROWS_T, _ROWS_T)],
                        out_hbm.at[pl.ds(c * N + s * _ROWS_T, _ROWS_T)])


def _agg_kernel(gsrc, dst, inv, y, zf):
    mesh = plsc.VectorSubcoreMesh(core_axis_name="c", subcore_axis_name="s")
    f = pl.kernel(
        _agg_body,
        out_type=jax.ShapeDtypeStruct((2 * N, D), jnp.float32),
        mesh=mesh,
        compiler_params=pltpu.CompilerParams(needs_layout_passes=False),
        scratch_types=[
            pltpu.VMEM_SHARED((N, D), jnp.float32),
            pltpu.VMEM((_EA,), jnp.int32),
            pltpu.VMEM((_EA,), jnp.int32),
            pltpu.VMEM((_EA,), jnp.float32),
            pltpu.VMEM((K, D), jnp.float32),
            pltpu.VMEM((K, D), jnp.float32),
            pltpu.SemaphoreType.DMA,
            pltpu.SemaphoreType.DMA,
            pltpu.SemaphoreType.DMA,
            pltpu.SemaphoreType.DMA,
        ],
    )
    return f(gsrc, dst, inv, y, zf)


def _layer(x, gsrc, dst, inv, zf, bases, comp, root, bias):
    wall = _w_matmul(comp, bases.reshape(NB, D * D))
    y = _y_matmul(x, wall.reshape(R * D, D))
    parts = _agg_kernel(gsrc, dst, inv, y, zf)
    return _final(parts, x, root, bias.reshape(1, D))


def kernel(edge_index, edge_type, emb, bases1, comp1, root1, bias1,
           bases2, comp2, root2, bias2):
    src = edge_index[0]
    dst = edge_index[1]
    gsrc = edge_type * N + src
    seg = edge_type * N + dst
    zf = jnp.zeros((N, D), jnp.float32)
    inv = _deg_kernel(seg)
    x1 = _layer(emb, gsrc, dst, inv, zf, bases1, comp1, root1, bias1)
    x2 = _layer(x1, gsrc, dst, inv, zf, bases2, comp2, root2, bias2)
    return x2
